# baseline (speedup 1.0000x reference)
import functools

import jax
import jax.numpy as jnp
from jax import lax
from jax.experimental import pallas as pl
from jax.experimental.pallas import tpu as pltpu
from jax.experimental.pallas import tpu_sc as plsc

N = 10000
F = 128
H = F // 2
E = 320000
NC = 2
NS = 16
NW = NC * NS
NP = 10240
RPT = NP // NS
C = 100
NCHUNK = E // C
CPTD = NCHUNK // NW
CPTS = NCHUNK // NS
R = 1000

_MESH = dict(core_axis_name="c", subcore_axis_name="s", num_cores=NC,
             num_subcores=NS)


@functools.partial(
    pl.kernel,
    out_type=jax.ShapeDtypeStruct((NC * NP,), jnp.float32),
    mesh=plsc.VectorSubcoreMesh(**_MESH),
    scratch_types=[
        pltpu.VMEM((CPTD, C), jnp.int32),
        pltpu.VMEM((RPT,), jnp.float32),
        pltpu.VMEM_SHARED((NP,), jnp.float32),
    ],
)
def _deg_kernel(row_hbm, out_hbm, row_v, ones_v, deg_sh):
    c = lax.axis_index("c")
    s = lax.axis_index("s")
    wid = c * NS + s
    pltpu.sync_copy(row_hbm.at[wid], row_v)

    def _fill(i, carry):
        ones_v[pl.ds(i * 16, 16)] = jnp.full((16,), 1.0, jnp.float32)
        return carry
    lax.fori_loop(0, RPT // 16, _fill, 0)
    pltpu.sync_copy(ones_v, deg_sh.at[pl.ds(s * RPT, RPT)])
    plsc.subcore_barrier()

    def _scat(j, carry):
        pltpu.sync_copy(ones_v.at[pl.ds(0, C)], deg_sh.at[row_v.at[j]],
                        add=True)
        return carry
    lax.fori_loop(0, CPTD, _scat, 0)
    plsc.subcore_barrier()
    pltpu.sync_copy(deg_sh.at[pl.ds(s * RPT, RPT)],
                    out_hbm.at[pl.ds(c * NP + s * RPT, RPT)])


@functools.partial(
    pl.kernel,
    out_type=jax.ShapeDtypeStruct((NC, NP, H), jnp.float32),
    mesh=plsc.VectorSubcoreMesh(**_MESH),
    scratch_types=[
        pltpu.VMEM((CPTS, C), jnp.int32),
        pltpu.VMEM((CPTS, C), jnp.int32),
        pltpu.VMEM((4, C, H), jnp.float32),
        pltpu.VMEM_SHARED((NP, H), jnp.float32),
        [pltpu.SemaphoreType.DMA] * 4,
        [pltpu.SemaphoreType.DMA] * 4,
    ],
    compiler_params=pltpu.CompilerParams(use_tc_tiling_on_sc=False),
)
def _spmm_kernel(hp_hbm, row_hbm, col_hbm, out_hbm,
                 row_v, col_v, ring, acc_sh, gsem, ssem):
    c = lax.axis_index("c")
    s = lax.axis_index("s")
    pltpu.sync_copy(row_hbm.at[s], row_v)
    pltpu.sync_copy(col_hbm.at[s], col_v)
    tab = hp_hbm.at[c]

    buf0 = ring.at[0]

    def _zrow(i, carry):
        for k in range(H // 16):
            ring[0, i, pl.ds(k * 16, 16)] = jnp.zeros((16,), jnp.float32)
        return carry
    lax.fori_loop(0, C, _zrow, 0)
    base = s * RPT
    nfull = RPT // C
    for t in range(nfull):
        pltpu.sync_copy(buf0, acc_sh.at[pl.ds(base + t * C, C)])
    rem = RPT - nfull * C
    pltpu.sync_copy(buf0.at[pl.ds(0, rem)],
                    acc_sh.at[pl.ds(base + nfull * C, rem)])
    plsc.subcore_barrier()

    for b in range(4):
        pltpu.async_copy(tab.at[row_v.at[b]], ring.at[b], gsem[b])

    def _step(t, carry):
        j0 = 4 * t
        for b in range(4):
            pltpu.make_async_copy(tab.at[row_v.at[j0 + b]], ring.at[b],
                                  gsem[b]).wait()
            pltpu.async_copy(ring.at[b], acc_sh.at[col_v.at[j0 + b]],
                             ssem[b], add=True)
        for b in range(4):
            pltpu.make_async_copy(ring.at[b], acc_sh.at[col_v.at[j0 + b]],
                                  ssem[b]).wait()

            @pl.when(j0 + b + 4 < CPTS)
            def _():
                pltpu.async_copy(tab.at[row_v.at[j0 + b + 4]], ring.at[b],
                                 gsem[b])
        return carry
    lax.fori_loop(0, CPTS // 4, _step, 0)
    plsc.subcore_barrier()
    pltpu.sync_copy(acc_sh.at[pl.ds(s * RPT, RPT)],
                    out_hbm.at[c].at[pl.ds(s * RPT, RPT)])


def _mm1_body(x_ref, w_ref, b_ref, h_ref):
    h_ref[...] = lax.dot_general(x_ref[...], w_ref[...],
                                 (((1,), (1,)), ((), ())),
                                 preferred_element_type=jnp.float32
                                 ) + b_ref[...]


def _scale_body(h_ref, d0_ref, d1_ref, hp_ref, dinv_ref):
    deg = d0_ref[0] + d1_ref[0] - 1.0
    dinv = lax.rsqrt(deg)
    hp = h_ref[...] * dinv
    hp_ref[0] = hp[:, :H]
    hp_ref[1] = hp[:, H:]
    dinv_ref[...] = dinv


def _mm2_body(p0_ref, p1_ref, h1_ref, dinv_ref, w_ref, b_ref,
              h_ref, hp_ref):
    dinv = dinv_ref[...]
    agg = jnp.concatenate([p0_ref[0], p1_ref[0]], axis=1)
    y = jnp.maximum(dinv * agg + (dinv * dinv) * h1_ref[...], 0.0)
    h = lax.dot_general(y, w_ref[...],
                        (((1,), (1,)), ((), ())),
                        preferred_element_type=jnp.float32) + b_ref[...]
    hp = h * dinv
    h_ref[...] = h
    hp_ref[0] = hp[:, :H]
    hp_ref[1] = hp[:, H:]


def _fin_body(q0_ref, q1_ref, h2_ref, dinv_ref, out_ref):
    dinv = dinv_ref[...]
    agg = jnp.concatenate([q0_ref[0], q1_ref[0]], axis=1)
    out_ref[...] = jnp.maximum(dinv * agg + (dinv * dinv) * h2_ref[...], 0.0)


def _row_spec():
    return pl.BlockSpec((R, F), lambda i: (i, 0))


def _part_spec(j):
    return pl.BlockSpec((1, R, H), lambda i, j=j: (j, i, 0))


def _dpart_spec(j):
    return pl.BlockSpec((1, R, 1), lambda i, j=j: (j, i, 0))


def _hp_spec():
    return pl.BlockSpec((NC, R, H), lambda i: (0, i, 0))


def _w_spec():
    return pl.BlockSpec((F, F), lambda i: (0, 0))


def _b_spec():
    return pl.BlockSpec((1, F), lambda i: (0, 0))


def _dinv_spec():
    return pl.BlockSpec((R, 1), lambda i: (i, 0))


def _mm1(xf, W1, b1):
    return pl.pallas_call(
        _mm1_body,
        grid=(N // R,),
        in_specs=[_row_spec(), _w_spec(), _b_spec()],
        out_specs=_row_spec(),
        out_shape=jax.ShapeDtypeStruct((N, F), jnp.float32),
    )(xf, W1, b1)


def _scale(h1, dparts3):
    return pl.pallas_call(
        _scale_body,
        grid=(N // R,),
        in_specs=[_row_spec(), _dpart_spec(0), _dpart_spec(1)],
        out_specs=[_hp_spec(), _dinv_spec()],
        out_shape=[jax.ShapeDtypeStruct((NC, N, H), jnp.float32),
                   jax.ShapeDtypeStruct((N, 1), jnp.float32)],
    )(h1, dparts3, dparts3)


def _mm2(parts, h1, dinv, W2, b2):
    return pl.pallas_call(
        _mm2_body,
        grid=(N // R,),
        in_specs=[_part_spec(0), _part_spec(1), _row_spec(), _dinv_spec(),
                  _w_spec(), _b_spec()],
        out_specs=[_row_spec(), _hp_spec()],
        out_shape=[jax.ShapeDtypeStruct((N, F), jnp.float32),
                   jax.ShapeDtypeStruct((NC, N, H), jnp.float32)],
    )(parts, parts, h1, dinv, W2, b2)


def _fin(parts, h2, dinv):
    return pl.pallas_call(
        _fin_body,
        grid=(N // R,),
        in_specs=[_part_spec(0), _part_spec(1), _row_spec(), _dinv_spec()],
        out_specs=_row_spec(),
        out_shape=jax.ShapeDtypeStruct((N, F), jnp.float32),
    )(parts, parts, h2, dinv)


@jax.jit
def _run(x, edge_index, W1, b1, W2, b2):
    xf = x[0]
    ei = edge_index.astype(jnp.int32)
    row_d = ei[0].reshape(NW, CPTD, C)
    row_s = ei[0].reshape(NS, CPTS, C)
    col_s = ei[1].reshape(NS, CPTS, C)

    dparts = _deg_kernel(row_d)
    dparts3 = dparts.reshape(NC, NP, 1)

    h1 = _mm1(xf, W1, b1.reshape(1, F))
    hp1, dinv = _scale(h1, dparts3)
    parts1 = _spmm_kernel(hp1, row_s, col_s)
    h2, hp2 = _mm2(parts1, h1, dinv, W2, b2.reshape(1, F))
    parts2 = _spmm_kernel(hp2, row_s, col_s)
    out = _fin(parts2, h2, dinv)
    return out[None]


def kernel(x, edge_index, W1, b1, W2, b2):
    return _run(x, edge_index, W1, b1, W2, b2)


# speedup vs baseline: 1.0336x; 1.0336x over previous
import functools

import jax
import jax.numpy as jnp
from jax import lax
from jax.experimental import pallas as pl
from jax.experimental.pallas import tpu as pltpu
from jax.experimental.pallas import tpu_sc as plsc

N = 10000
F = 128
H = F // 2
E = 320000
NC = 2
NS = 16
NW = NC * NS
NP = 10240
RPT = NP // NS
C = 125
NCHUNK = E // C
CPTD = NCHUNK // NW
CPTS = NCHUNK // NS
R = 1000

_MESH = dict(core_axis_name="c", subcore_axis_name="s", num_cores=NC,
             num_subcores=NS)


@functools.partial(
    pl.kernel,
    out_type=jax.ShapeDtypeStruct((NC * NP,), jnp.float32),
    mesh=plsc.VectorSubcoreMesh(**_MESH),
    scratch_types=[
        pltpu.VMEM((CPTD, C), jnp.int32),
        pltpu.VMEM((RPT,), jnp.float32),
        pltpu.VMEM_SHARED((NP,), jnp.float32),
    ],
)
def _deg_kernel(row_hbm, out_hbm, row_v, ones_v, deg_sh):
    c = lax.axis_index("c")
    s = lax.axis_index("s")
    wid = c * NS + s
    pltpu.sync_copy(row_hbm.at[wid], row_v)

    def _fill(i, carry):
        ones_v[pl.ds(i * 16, 16)] = jnp.full((16,), 1.0, jnp.float32)
        return carry
    lax.fori_loop(0, RPT // 16, _fill, 0)
    pltpu.sync_copy(ones_v, deg_sh.at[pl.ds(s * RPT, RPT)])
    plsc.subcore_barrier()

    def _scat(j, carry):
        pltpu.sync_copy(ones_v.at[pl.ds(0, C)], deg_sh.at[row_v.at[j]],
                        add=True)
        return carry
    lax.fori_loop(0, CPTD, _scat, 0)
    plsc.subcore_barrier()
    pltpu.sync_copy(deg_sh.at[pl.ds(s * RPT, RPT)],
                    out_hbm.at[pl.ds(c * NP + s * RPT, RPT)])


@functools.partial(
    pl.kernel,
    out_type=jax.ShapeDtypeStruct((NC, NP, H), jnp.float32),
    mesh=plsc.VectorSubcoreMesh(**_MESH),
    scratch_types=[
        pltpu.VMEM((CPTS, C), jnp.int32),
        pltpu.VMEM((CPTS, C), jnp.int32),
        pltpu.VMEM((4, C, H), jnp.float32),
        pltpu.VMEM_SHARED((NP, H), jnp.float32),
        [pltpu.SemaphoreType.DMA] * 4,
        [pltpu.SemaphoreType.DMA] * 4,
    ],
    compiler_params=pltpu.CompilerParams(use_tc_tiling_on_sc=False),
)
def _spmm_kernel(hp_hbm, row_hbm, col_hbm, out_hbm,
                 row_v, col_v, ring, acc_sh, gsem, ssem):
    c = lax.axis_index("c")
    s = lax.axis_index("s")
    pltpu.sync_copy(row_hbm.at[s], row_v)
    pltpu.sync_copy(col_hbm.at[s], col_v)
    tab = hp_hbm.at[c]

    buf0 = ring.at[0]

    def _zrow(i, carry):
        for k in range(H // 16):
            ring[0, i, pl.ds(k * 16, 16)] = jnp.zeros((16,), jnp.float32)
        return carry
    lax.fori_loop(0, C, _zrow, 0)
    base = s * RPT
    nfull = RPT // C
    for t in range(nfull):
        pltpu.sync_copy(buf0, acc_sh.at[pl.ds(base + t * C, C)])
    rem = RPT - nfull * C
    pltpu.sync_copy(buf0.at[pl.ds(0, rem)],
                    acc_sh.at[pl.ds(base + nfull * C, rem)])
    plsc.subcore_barrier()

    for b in range(4):
        pltpu.async_copy(tab.at[row_v.at[b]], ring.at[b], gsem[b])

    def _step(t, carry):
        j0 = 4 * t
        for b in range(4):
            pltpu.make_async_copy(tab.at[row_v.at[j0 + b]], ring.at[b],
                                  gsem[b]).wait()
            pltpu.async_copy(ring.at[b], acc_sh.at[col_v.at[j0 + b]],
                             ssem[b], add=True)
        for b in range(4):
            pltpu.make_async_copy(ring.at[b], acc_sh.at[col_v.at[j0 + b]],
                                  ssem[b]).wait()

            @pl.when(j0 + b + 4 < CPTS)
            def _():
                pltpu.async_copy(tab.at[row_v.at[j0 + b + 4]], ring.at[b],
                                 gsem[b])
        return carry
    lax.fori_loop(0, CPTS // 4, _step, 0)
    plsc.subcore_barrier()
    pltpu.sync_copy(acc_sh.at[pl.ds(s * RPT, RPT)],
                    out_hbm.at[c].at[pl.ds(s * RPT, RPT)])


def _mm1_body(x_ref, w_ref, b_ref, d0_ref, d1_ref,
              h_ref, hp_ref, dinv_ref):
    deg = d0_ref[0] + d1_ref[0] - 1.0
    dinv = lax.rsqrt(deg)
    h = lax.dot_general(x_ref[...], w_ref[...],
                        (((1,), (1,)), ((), ())),
                        preferred_element_type=jnp.float32) + b_ref[...]
    hp = h * dinv
    h_ref[...] = h
    hp_ref[0] = hp[:, :H]
    hp_ref[1] = hp[:, H:]
    dinv_ref[...] = dinv


def _mm2_body(p0_ref, p1_ref, h1_ref, dinv_ref, w_ref, b_ref,
              h_ref, hp_ref):
    dinv = dinv_ref[...]
    agg = jnp.concatenate([p0_ref[0], p1_ref[0]], axis=1)
    y = jnp.maximum(dinv * agg + (dinv * dinv) * h1_ref[...], 0.0)
    h = lax.dot_general(y, w_ref[...],
                        (((1,), (1,)), ((), ())),
                        preferred_element_type=jnp.float32) + b_ref[...]
    hp = h * dinv
    h_ref[...] = h
    hp_ref[0] = hp[:, :H]
    hp_ref[1] = hp[:, H:]


def _fin_body(q0_ref, q1_ref, h2_ref, dinv_ref, out_ref):
    dinv = dinv_ref[...]
    agg = jnp.concatenate([q0_ref[0], q1_ref[0]], axis=1)
    out_ref[...] = jnp.maximum(dinv * agg + (dinv * dinv) * h2_ref[...], 0.0)


def _row_spec():
    return pl.BlockSpec((R, F), lambda i: (i, 0))


def _part_spec(j):
    return pl.BlockSpec((1, R, H), lambda i, j=j: (j, i, 0))


def _dpart_spec(j):
    return pl.BlockSpec((1, R, 1), lambda i, j=j: (j, i, 0))


def _hp_spec():
    return pl.BlockSpec((NC, R, H), lambda i: (0, i, 0))


def _w_spec():
    return pl.BlockSpec((F, F), lambda i: (0, 0))


def _b_spec():
    return pl.BlockSpec((1, F), lambda i: (0, 0))


def _dinv_spec():
    return pl.BlockSpec((R, 1), lambda i: (i, 0))


def _mm1(xf, W1, b1, dparts3):
    return pl.pallas_call(
        _mm1_body,
        grid=(N // R,),
        in_specs=[_row_spec(), _w_spec(), _b_spec(),
                  _dpart_spec(0), _dpart_spec(1)],
        out_specs=[_row_spec(), _hp_spec(), _dinv_spec()],
        out_shape=[jax.ShapeDtypeStruct((N, F), jnp.float32),
                   jax.ShapeDtypeStruct((NC, N, H), jnp.float32),
                   jax.ShapeDtypeStruct((N, 1), jnp.float32)],
    )(xf, W1, b1, dparts3, dparts3)


def _mm2(parts, h1, dinv, W2, b2):
    return pl.pallas_call(
        _mm2_body,
        grid=(N // R,),
        in_specs=[_part_spec(0), _part_spec(1), _row_spec(), _dinv_spec(),
                  _w_spec(), _b_spec()],
        out_specs=[_row_spec(), _hp_spec()],
        out_shape=[jax.ShapeDtypeStruct((N, F), jnp.float32),
                   jax.ShapeDtypeStruct((NC, N, H), jnp.float32)],
    )(parts, parts, h1, dinv, W2, b2)


def _fin(parts, h2, dinv):
    return pl.pallas_call(
        _fin_body,
        grid=(N // R,),
        in_specs=[_part_spec(0), _part_spec(1), _row_spec(), _dinv_spec()],
        out_specs=_row_spec(),
        out_shape=jax.ShapeDtypeStruct((N, F), jnp.float32),
    )(parts, parts, h2, dinv)


@jax.jit
def _run(x, edge_index, W1, b1, W2, b2):
    xf = x[0]
    ei = edge_index.astype(jnp.int32)
    row_d = ei[0].reshape(NW, CPTD, C)
    row_s = ei[0].reshape(NS, CPTS, C)
    col_s = ei[1].reshape(NS, CPTS, C)

    dparts = _deg_kernel(row_d)
    dparts3 = dparts.reshape(NC, NP, 1)

    h1, hp1, dinv = _mm1(xf, W1, b1.reshape(1, F), dparts3)
    parts1 = _spmm_kernel(hp1, row_s, col_s)
    h2, hp2 = _mm2(parts1, h1, dinv, W2, b2.reshape(1, F))
    parts2 = _spmm_kernel(hp2, row_s, col_s)
    out = _fin(parts2, h2, dinv)
    return out[None]


def kernel(x, edge_index, W1, b1, W2, b2):
    return _run(x, edge_index, W1, b1, W2, b2)


# speedup vs baseline: 1.0476x; 1.0135x over previous
import functools

import jax
import jax.numpy as jnp
from jax import lax
from jax.experimental import pallas as pl
from jax.experimental.pallas import tpu as pltpu
from jax.experimental.pallas import tpu_sc as plsc

N = 10000
F = 128
H = F // 2
E = 320000
NC = 2
NS = 16
NW = NC * NS
NP = 10240
RPT = NP // NS
C = 125
NCHUNK = E // C
CPTD = NCHUNK // NW
CPTS = NCHUNK // NS
R = 1000
NBUF = 5

_MESH = dict(core_axis_name="c", subcore_axis_name="s", num_cores=NC,
             num_subcores=NS)


@functools.partial(
    pl.kernel,
    out_type=jax.ShapeDtypeStruct((NC * NP,), jnp.float32),
    mesh=plsc.VectorSubcoreMesh(**_MESH),
    scratch_types=[
        pltpu.VMEM((CPTD, C), jnp.int32),
        pltpu.VMEM((RPT,), jnp.float32),
        pltpu.VMEM_SHARED((NP,), jnp.float32),
    ],
)
def _deg_kernel(row_hbm, out_hbm, row_v, ones_v, deg_sh):
    c = lax.axis_index("c")
    s = lax.axis_index("s")
    wid = c * NS + s
    pltpu.sync_copy(row_hbm.at[wid], row_v)

    def _fill(i, carry):
        ones_v[pl.ds(i * 16, 16)] = jnp.full((16,), 1.0, jnp.float32)
        return carry
    lax.fori_loop(0, RPT // 16, _fill, 0)
    pltpu.sync_copy(ones_v, deg_sh.at[pl.ds(s * RPT, RPT)])
    plsc.subcore_barrier()

    def _scat(j, carry):
        pltpu.sync_copy(ones_v.at[pl.ds(0, C)], deg_sh.at[row_v.at[j]],
                        add=True)
        return carry
    lax.fori_loop(0, CPTD, _scat, 0)
    plsc.subcore_barrier()
    pltpu.sync_copy(deg_sh.at[pl.ds(s * RPT, RPT)],
                    out_hbm.at[pl.ds(c * NP + s * RPT, RPT)])


@functools.partial(
    pl.kernel,
    out_type=jax.ShapeDtypeStruct((NC, NP, H), jnp.float32),
    mesh=plsc.VectorSubcoreMesh(**_MESH),
    scratch_types=[
        pltpu.VMEM((CPTS, C), jnp.int32),
        pltpu.VMEM((CPTS, C), jnp.int32),
        pltpu.VMEM((NBUF, C, H), jnp.float32),
        pltpu.VMEM_SHARED((NP, H), jnp.float32),
        [pltpu.SemaphoreType.DMA] * NBUF,
        [pltpu.SemaphoreType.DMA] * NBUF,
    ],
    compiler_params=pltpu.CompilerParams(use_tc_tiling_on_sc=False),
)
def _spmm_kernel(hp_hbm, row_hbm, col_hbm, out_hbm,
                 row_v, col_v, ring, acc_sh, gsem, ssem):
    c = lax.axis_index("c")
    s = lax.axis_index("s")
    pltpu.sync_copy(row_hbm.at[s], row_v)
    pltpu.sync_copy(col_hbm.at[s], col_v)
    tab = hp_hbm.at[c]

    buf0 = ring.at[0]

    def _zrow(i, carry):
        for k in range(H // 16):
            ring[0, i, pl.ds(k * 16, 16)] = jnp.zeros((16,), jnp.float32)
        return carry
    lax.fori_loop(0, C, _zrow, 0)
    base = s * RPT
    nfull = RPT // C
    for t in range(nfull):
        pltpu.sync_copy(buf0, acc_sh.at[pl.ds(base + t * C, C)])
    rem = RPT - nfull * C
    pltpu.sync_copy(buf0.at[pl.ds(0, rem)],
                    acc_sh.at[pl.ds(base + nfull * C, rem)])
    plsc.subcore_barrier()

    for b in range(NBUF):
        pltpu.async_copy(tab.at[row_v.at[b]], ring.at[b], gsem[b])

    def _step(t, carry):
        j0 = NBUF * t
        for b in range(NBUF):
            pltpu.make_async_copy(tab.at[row_v.at[j0 + b]], ring.at[b],
                                  gsem[b]).wait()
            pltpu.async_copy(ring.at[b], acc_sh.at[col_v.at[j0 + b]],
                             ssem[b], add=True)
        for b in range(NBUF):
            pltpu.make_async_copy(ring.at[b], acc_sh.at[col_v.at[j0 + b]],
                                  ssem[b]).wait()

            @pl.when(j0 + b + NBUF < CPTS)
            def _():
                pltpu.async_copy(tab.at[row_v.at[j0 + b + NBUF]], ring.at[b],
                                 gsem[b])
        return carry
    lax.fori_loop(0, CPTS // NBUF, _step, 0)
    plsc.subcore_barrier()
    pltpu.sync_copy(acc_sh.at[pl.ds(s * RPT, RPT)],
                    out_hbm.at[c].at[pl.ds(s * RPT, RPT)])


def _mm1_body(x_ref, w_ref, b_ref, d0_ref, d1_ref,
              h_ref, hp_ref, dinv_ref):
    deg = d0_ref[0] + d1_ref[0] - 1.0
    dinv = lax.rsqrt(deg)
    h = lax.dot_general(x_ref[...], w_ref[...],
                        (((1,), (1,)), ((), ())),
                        preferred_element_type=jnp.float32) + b_ref[...]
    hp = h * dinv
    h_ref[...] = h
    hp_ref[0] = hp[:, :H]
    hp_ref[1] = hp[:, H:]
    dinv_ref[...] = dinv


def _mm2_body(p0_ref, p1_ref, h1_ref, dinv_ref, w_ref, b_ref,
              h_ref, hp_ref):
    dinv = dinv_ref[...]
    agg = jnp.concatenate([p0_ref[0], p1_ref[0]], axis=1)
    y = jnp.maximum(dinv * agg + (dinv * dinv) * h1_ref[...], 0.0)
    h = lax.dot_general(y, w_ref[...],
                        (((1,), (1,)), ((), ())),
                        preferred_element_type=jnp.float32) + b_ref[...]
    hp = h * dinv
    h_ref[...] = h
    hp_ref[0] = hp[:, :H]
    hp_ref[1] = hp[:, H:]


def _fin_body(q0_ref, q1_ref, h2_ref, dinv_ref, out_ref):
    dinv = dinv_ref[...]
    agg = jnp.concatenate([q0_ref[0], q1_ref[0]], axis=1)
    out_ref[...] = jnp.maximum(dinv * agg + (dinv * dinv) * h2_ref[...], 0.0)


def _row_spec():
    return pl.BlockSpec((R, F), lambda i: (i, 0))


def _part_spec(j):
    return pl.BlockSpec((1, R, H), lambda i, j=j: (j, i, 0))


def _dpart_spec(j):
    return pl.BlockSpec((1, R, 1), lambda i, j=j: (j, i, 0))


def _hp_spec():
    return pl.BlockSpec((NC, R, H), lambda i: (0, i, 0))


def _w_spec():
    return pl.BlockSpec((F, F), lambda i: (0, 0))


def _b_spec():
    return pl.BlockSpec((1, F), lambda i: (0, 0))


def _dinv_spec():
    return pl.BlockSpec((R, 1), lambda i: (i, 0))


def _mm1(xf, W1, b1, dparts3):
    return pl.pallas_call(
        _mm1_body,
        grid=(N // R,),
        in_specs=[_row_spec(), _w_spec(), _b_spec(),
                  _dpart_spec(0), _dpart_spec(1)],
        out_specs=[_row_spec(), _hp_spec(), _dinv_spec()],
        out_shape=[jax.ShapeDtypeStruct((N, F), jnp.float32),
                   jax.ShapeDtypeStruct((NC, N, H), jnp.float32),
                   jax.ShapeDtypeStruct((N, 1), jnp.float32)],
    )(xf, W1, b1, dparts3, dparts3)


def _mm2(parts, h1, dinv, W2, b2):
    return pl.pallas_call(
        _mm2_body,
        grid=(N // R,),
        in_specs=[_part_spec(0), _part_spec(1), _row_spec(), _dinv_spec(),
                  _w_spec(), _b_spec()],
        out_specs=[_row_spec(), _hp_spec()],
        out_shape=[jax.ShapeDtypeStruct((N, F), jnp.float32),
                   jax.ShapeDtypeStruct((NC, N, H), jnp.float32)],
    )(parts, parts, h1, dinv, W2, b2)


def _fin(parts, h2, dinv):
    return pl.pallas_call(
        _fin_body,
        grid=(N // R,),
        in_specs=[_part_spec(0), _part_spec(1), _row_spec(), _dinv_spec()],
        out_specs=_row_spec(),
        out_shape=jax.ShapeDtypeStruct((N, F), jnp.float32),
    )(parts, parts, h2, dinv)


@jax.jit
def _run(x, edge_index, W1, b1, W2, b2):
    xf = x[0]
    ei = edge_index.astype(jnp.int32)
    row_d = ei[0].reshape(NW, CPTD, C)
    row_s = ei[0].reshape(NS, CPTS, C)
    col_s = ei[1].reshape(NS, CPTS, C)

    dparts = _deg_kernel(row_d)
    dparts3 = dparts.reshape(NC, NP, 1)

    h1, hp1, dinv = _mm1(xf, W1, b1.reshape(1, F), dparts3)
    parts1 = _spmm_kernel(hp1, row_s, col_s)
    h2, hp2 = _mm2(parts1, h1, dinv, W2, b2.reshape(1, F))
    parts2 = _spmm_kernel(hp2, row_s, col_s)
    out = _fin(parts2, h2, dinv)
    return out[None]


def kernel(x, edge_index, W1, b1, W2, b2):
    return _run(x, edge_index, W1, b1, W2, b2)
